# baseline (device time: 39323 ns/iter reference)
import jax
import jax.numpy as jnp
from jax import lax
from jax.experimental import pallas as pl
from jax.experimental.pallas import tpu as pltpu


def kernel(Q, K, V):
    b, sq, h, d = Q.shape
    skv = K.shape[1]
    scale = d ** -0.5

    def body(q_ref, k_ref, v_ref, out_ref, comm_ref, send_sem, recv_sem):
        my_x = lax.axis_index("x")
        my_y = lax.axis_index("y")
        my_z = lax.axis_index("z")
        partner = (my_x, my_y, 1 - my_z)

        barrier_sem = pltpu.get_barrier_semaphore()
        pl.semaphore_signal(
            barrier_sem, inc=1,
            device_id=partner, device_id_type=pl.DeviceIdType.MESH,
        )
        pl.semaphore_wait(barrier_sem, 1)

        q = q_ref[...][:, 0, :, :]
        k = k_ref[...]
        v = v_ref[...]
        s = jnp.sum(q[:, None, :, :] * k, axis=3) * scale
        m = jnp.max(s, axis=1)
        p = jnp.exp(s - m[:, None, :])
        l = jnp.sum(p, axis=1)
        o = jnp.sum(p[:, :, :, None] * v, axis=1)

        comm_ref[0, :, :, 0:d] = o
        comm_ref[0, :, :, d:d + 1] = m[:, :, None]
        comm_ref[0, :, :, d + 1:d + 2] = l[:, :, None]

        rdma = pltpu.make_async_remote_copy(
            src_ref=comm_ref.at[0],
            dst_ref=comm_ref.at[1],
            send_sem=send_sem,
            recv_sem=recv_sem,
            device_id=partner,
            device_id_type=pl.DeviceIdType.MESH,
        )
        rdma.start()
        rdma.wait()

        o2 = comm_ref[1, :, :, 0:d]
        m2 = comm_ref[1, :, :, d:d + 1]
        l2 = comm_ref[1, :, :, d + 1:d + 2]
        m1 = m[:, :, None]
        l1 = l[:, :, None]
        mn = jnp.maximum(m1, m2)
        a1 = jnp.exp(m1 - mn)
        a2 = jnp.exp(m2 - mn)
        ln = a1 * l1 + a2 * l2
        on = (a1 * o + a2 * o2) / ln
        out_ref[...] = on[:, None, :, :]

    return pl.pallas_call(
        body,
        out_shape=jax.ShapeDtypeStruct((b, sq, h, d), jnp.float32),
        in_specs=[
            pl.BlockSpec(memory_space=pltpu.VMEM),
            pl.BlockSpec(memory_space=pltpu.VMEM),
            pl.BlockSpec(memory_space=pltpu.VMEM),
        ],
        out_specs=pl.BlockSpec(memory_space=pltpu.VMEM),
        scratch_shapes=[
            pltpu.VMEM((2, b, h, d + 2), jnp.float32),
            pltpu.SemaphoreType.DMA,
            pltpu.SemaphoreType.DMA,
        ],
        compiler_params=pltpu.CompilerParams(collective_id=0),
    )(Q, K, V)


# device time: 14556 ns/iter; 2.7015x vs baseline; 2.7015x over previous
import jax
import jax.numpy as jnp
from jax import lax
from jax.experimental import pallas as pl
from jax.experimental.pallas import tpu as pltpu


def kernel(Q, K, V):
    b, sq, h, d = Q.shape
    skv = K.shape[1]
    scale = d ** -0.5

    Kt = jnp.transpose(K, (0, 2, 3, 1))
    Vt = jnp.transpose(V, (0, 2, 3, 1))
    Qs = Q[:, 0, :, :]

    def body(q_ref, k_ref, v_ref, out_ref, comm_ref, send_sem, recv_sem):
        my_x = lax.axis_index("x")
        my_y = lax.axis_index("y")
        my_z = lax.axis_index("z")
        partner = (my_x, my_y, 1 - my_z)

        barrier_sem = pltpu.get_barrier_semaphore()
        pl.semaphore_signal(
            barrier_sem, inc=1,
            device_id=partner, device_id_type=pl.DeviceIdType.MESH,
        )
        pl.semaphore_wait(barrier_sem, 1)

        q = q_ref[...]
        k = k_ref[...]
        v = v_ref[...]
        s = jnp.sum(q[:, :, :, None] * k, axis=2) * scale
        m = jnp.max(s, axis=2, keepdims=True)
        p = jnp.exp(s - m)
        l = jnp.sum(p, axis=2, keepdims=True)
        o = jnp.sum(p[:, :, None, :] * v, axis=3)

        comm_ref[0, :, :, 0:d] = o
        comm_ref[0, :, :, d:d + 1] = m
        comm_ref[0, :, :, d + 1:d + 2] = l

        rdma = pltpu.make_async_remote_copy(
            src_ref=comm_ref.at[0],
            dst_ref=comm_ref.at[1],
            send_sem=send_sem,
            recv_sem=recv_sem,
            device_id=partner,
            device_id_type=pl.DeviceIdType.MESH,
        )
        rdma.start()
        rdma.wait()

        o2 = comm_ref[1, :, :, 0:d]
        m2 = comm_ref[1, :, :, d:d + 1]
        l2 = comm_ref[1, :, :, d + 1:d + 2]
        mn = jnp.maximum(m, m2)
        a1 = jnp.exp(m - mn)
        a2 = jnp.exp(m2 - mn)
        ln = a1 * l + a2 * l2
        on = (a1 * o + a2 * o2) / ln
        out_ref[...] = on[:, None, :, :]

    return pl.pallas_call(
        body,
        out_shape=jax.ShapeDtypeStruct((b, sq, h, d), jnp.float32),
        in_specs=[
            pl.BlockSpec(memory_space=pltpu.VMEM),
            pl.BlockSpec(memory_space=pltpu.VMEM),
            pl.BlockSpec(memory_space=pltpu.VMEM),
        ],
        out_specs=pl.BlockSpec(memory_space=pltpu.VMEM),
        scratch_shapes=[
            pltpu.VMEM((2, b, h, d + 2), jnp.float32),
            pltpu.SemaphoreType.DMA,
            pltpu.SemaphoreType.DMA,
        ],
        compiler_params=pltpu.CompilerParams(collective_id=0),
    )(Qs, Kt, Vt)
